# baseline (speedup 1.0000x reference)
import functools

import jax
import jax.numpy as jnp
from jax import lax
from jax.experimental import pallas as pl
from jax.experimental.pallas import tpu as pltpu
from jax.experimental.pallas import tpu_sc as plsc

_S, _B, _D = 2048, 4, 1024
_L = 16
_NC, _NS = 2, 16
_NW = _NC * _NS
_RPW = _S // _NW
_R = 8
_NCH = _RPW // _R
_BD = _B * _D
_XC = _R * _BD
_PC = _R * _D


def _sc_body(x_hbm, pos_hbm, out_hbm, xbuf, pbuf, sx0, sx1, sp0, sp1, so0, so1):
    wid = lax.axis_index("s") * _NC + lax.axis_index("c")
    xbase = wid * (_RPW * _BD)
    pbase = wid * (_RPW * _D)
    sx = (sx0, sx1)
    sp = (sp0, sp1)
    so = (so0, so1)
    loads = [None, None]
    stores = [None, None]

    def start_load(g):
        b = g % 2
        cx = pltpu.async_copy(
            x_hbm.at[pl.ds(xbase + g * _XC, _XC)], xbuf.at[b], sx[b])
        cp = pltpu.async_copy(
            pos_hbm.at[pl.ds(pbase + g * _PC, _PC)], pbuf.at[b], sp[b])
        loads[b] = (cx, cp)

    start_load(0)
    for g in range(_NCH):
        b = g % 2
        if g + 1 < _NCH:
            if stores[1 - b] is not None:
                stores[1 - b].wait()
            start_load(g + 1)
        cx, cp = loads[b]
        cx.wait()
        cp.wait()

        @plsc.parallel_loop(0, _PC, step=_L, unroll=8)
        def _accumulate(q, _b=b):
            q = pl.multiple_of(q, _L)
            pvec = pbuf[_b, pl.ds(q, _L)]
            xo = pl.multiple_of(q + 3 * (q & (-_D)), _L)
            for bb in range(_B):
                plsc.addupdate(xbuf.at[_b, pl.ds(xo + bb * _D, _L)], pvec)

        stores[b] = pltpu.async_copy(
            xbuf.at[b], out_hbm.at[pl.ds(xbase + g * _XC, _XC)], so[b])
    stores[0].wait()
    stores[1].wait()


@jax.jit
def _sc_add(x1d, pos1d):
    run = pl.kernel(
        _sc_body,
        out_type=jax.ShapeDtypeStruct((_S * _BD,), jnp.float32),
        mesh=plsc.VectorSubcoreMesh(
            core_axis_name="c", subcore_axis_name="s",
            num_cores=_NC, num_subcores=_NS),
        scratch_types=[
            pltpu.VMEM((2, _XC), jnp.float32),
            pltpu.VMEM((2, _PC), jnp.float32),
            pltpu.SemaphoreType.DMA,
            pltpu.SemaphoreType.DMA,
            pltpu.SemaphoreType.DMA,
            pltpu.SemaphoreType.DMA,
            pltpu.SemaphoreType.DMA,
            pltpu.SemaphoreType.DMA,
        ],
    )
    return run(x1d, pos1d)


def kernel(x, pos_table):
    S, B, D = x.shape
    out = _sc_add(x.reshape(S * B * D), pos_table[:S].reshape(S * D))
    return out.reshape(S, B, D)


# speedup vs baseline: 4.7419x; 4.7419x over previous
import jax
import jax.numpy as jnp
from jax.experimental import pallas as pl
from jax.experimental.pallas import tpu as pltpu


_BS = 128


def _add_body(x_ref, pos_ref, o_ref):
    o_ref[...] = x_ref[...] + pos_ref[...][:, None, :]


def kernel(x, pos_table):
    S, B, D = x.shape
    grid = (S // _BS,)
    return pl.pallas_call(
        _add_body,
        grid=grid,
        in_specs=[
            pl.BlockSpec((_BS, B, D), lambda i: (i, 0, 0)),
            pl.BlockSpec((_BS, D), lambda i: (i, 0)),
        ],
        out_specs=pl.BlockSpec((_BS, B, D), lambda i: (i, 0, 0)),
        out_shape=jax.ShapeDtypeStruct((S, B, D), x.dtype),
    )(x, pos_table[:S])


# speedup vs baseline: 5.2073x; 1.0982x over previous
import jax
import jax.numpy as jnp
from jax.experimental import pallas as pl
from jax.experimental.pallas import tpu as pltpu


_BS = 512


def _add_body(x_ref, pos_ref, o_ref):
    o_ref[...] = x_ref[...] + pos_ref[...][:, None, :]


def kernel(x, pos_table):
    S, B, D = x.shape
    grid = (S // _BS,)
    return pl.pallas_call(
        _add_body,
        grid=grid,
        in_specs=[
            pl.BlockSpec((_BS, B, D), lambda i: (i, 0, 0)),
            pl.BlockSpec((_BS, D), lambda i: (i, 0)),
        ],
        out_specs=pl.BlockSpec((_BS, B, D), lambda i: (i, 0, 0)),
        out_shape=jax.ShapeDtypeStruct((S, B, D), x.dtype),
    )(x, pos_table[:S])
